# baseline (speedup 1.0000x reference)
import jax
import jax.numpy as jnp
from jax import lax
from jax.experimental import pallas as pl
from jax.experimental.pallas import tpu as pltpu
from jax.experimental.pallas import tpu_sc as plsc

_L = 16
_NC = 2
_NS = 16
_NW = _NC * _NS

_BATCH = 64
_NIDX = 64
_DIM = 3
_ATM = 100000
_BROW = _ATM * _DIM

_BATCH_PER_W = _BATCH // _NW
_ELEM_PER_B = _NIDX * _DIM
_ELEM_PER_W = _BATCH_PER_W * _ELEM_PER_B
_CHUNKS_PER_B = _ELEM_PER_B // _L
_NGATHER = _ELEM_PER_W // 128


def _gather_body(pos_hbm, idx_hbm, out_hbm, idx_v, eidx_v, rows_v, sem):
    wid = lax.axis_index("s") * _NC + lax.axis_index("c")
    pltpu.sync_copy(idx_hbm, idx_v)
    base_b = wid * _BATCH_PER_W
    lanes = jnp.arange(_L, dtype=jnp.int32)
    for jc in range(_NIDX // _L):
        jvec = idx_v[pl.ds(jc * _L, _L)]
        for m in range(_DIM):
            r = lanes + m * _L
            jl = lax.div(r, jnp.int32(_DIM))
            d = r - jl * _DIM
            dnums = lax.GatherDimensionNumbers(
                offset_dims=(), collapsed_slice_dims=(0,), start_index_map=(0,)
            )
            src = lax.gather(
                jvec, jl[:, None], dnums, (1,),
                mode=lax.GatherScatterMode.PROMISE_IN_BOUNDS,
            )
            pat = src * _DIM + d
            for t in range(_BATCH_PER_W):
                boff = (base_b + t) * _BROW
                cc = t * _CHUNKS_PER_B + jc * _DIM + m
                eidx_v[cc // 8, pl.ds((cc % 8) * _L, _L)] = pat + boff
    copies = [
        pltpu.async_copy(pos_hbm.at[eidx_v.at[g]], rows_v.at[g], sem)
        for g in range(_NGATHER)
    ]
    for cp in copies:
        cp.wait()
    for g in range(_NGATHER):
        pltpu.sync_copy(
            rows_v.at[g], out_hbm.at[pl.ds(wid * _ELEM_PER_W + g * 128, 128)]
        )


@jax.jit
def _fixgen_gather(pos_flat, idx32):
    mesh = plsc.VectorSubcoreMesh(core_axis_name="c", subcore_axis_name="s")
    run = pl.kernel(
        _gather_body,
        mesh=mesh,
        out_type=jax.ShapeDtypeStruct((_BATCH * _ELEM_PER_B,), jnp.float32),
        scratch_types=[
            pltpu.VMEM((_NIDX,), jnp.int32),
            pltpu.VMEM((_NGATHER, 128), jnp.int32),
            pltpu.VMEM((_NGATHER, 128), jnp.float32),
            pltpu.SemaphoreType.DMA,
        ],
    )
    return run(pos_flat, idx32)


def kernel(pos, idx):
    batch, atm, dim = pos.shape
    pos_flat = pos.reshape(batch * atm * dim)
    idx32 = idx.astype(jnp.int32)
    out = _fixgen_gather(pos_flat, idx32)
    return out.reshape(batch, idx.shape[0] * dim)


# speedup vs baseline: 1.0019x; 1.0019x over previous
import jax
import jax.numpy as jnp
from jax import lax
from jax.experimental import pallas as pl
from jax.experimental.pallas import tpu as pltpu
from jax.experimental.pallas import tpu_sc as plsc

_NC = 2
_NS = 16
_NW = _NC * _NS

_BATCH = 64
_NIDX = 64
_DIM = 3
_ATM = 100000

_BATCH_PER_W = _BATCH // _NW
_OUT_PER_W = _BATCH_PER_W * _NIDX * _DIM
_CH = 128
_NCH = _OUT_PER_W // _CH


def _gather_body(pos_hbm, eidx_hbm, out_hbm, eidx_v, vals_v, sem):
    wid = lax.axis_index("s") * _NC + lax.axis_index("c")
    pltpu.sync_copy(eidx_hbm.at[wid], eidx_v)
    cps = [
        pltpu.async_copy(pos_hbm.at[eidx_v.at[r]], vals_v.at[r], sem)
        for r in range(_NCH)
    ]
    for cp in cps:
        cp.wait()
    pltpu.sync_copy(vals_v, out_hbm.at[wid])


@jax.jit
def _fixgen_gather(pos_flat, eidx):
    mesh = plsc.VectorSubcoreMesh(core_axis_name="c", subcore_axis_name="s")
    run = pl.kernel(
        _gather_body,
        mesh=mesh,
        out_type=jax.ShapeDtypeStruct((_NW, _NCH, _CH), jnp.float32),
        scratch_types=[
            pltpu.VMEM((_NCH, _CH), jnp.int32),
            pltpu.VMEM((_NCH, _CH), jnp.float32),
            pltpu.SemaphoreType.DMA,
        ],
    )
    return run(pos_flat, eidx)


def kernel(pos, idx):
    batch, atm, dim = pos.shape
    idx32 = idx.astype(jnp.int32)
    ebase = (jnp.arange(batch, dtype=jnp.int32)[:, None] * atm
             + idx32[None, :]) * dim
    eidx = (ebase[:, :, None]
            + jnp.arange(dim, dtype=jnp.int32)).reshape(_NW, _NCH, _CH)
    out32 = _fixgen_gather(pos.reshape(batch * atm * dim), eidx)
    return out32.reshape(batch, idx.shape[0] * dim)


# speedup vs baseline: 13.3616x; 13.3367x over previous
import jax
import jax.numpy as jnp
from jax import lax
from jax.experimental import pallas as pl
from jax.experimental.pallas import tpu as pltpu

_BATCH = 64
_NIDX = 64
_DIM = 3
_ATM = 100000

_JPB = 8
_STEPS = _NIDX // _JPB


def _gather_body(idx_ref, *refs):
    x_refs = refs[:_JPB]
    o_ref = refs[_JPB]
    s = pl.program_id(0)
    sub = lax.broadcasted_iota(jnp.int32, (1, 8, 1), 1)
    for t in range(_JPB):
        j = idx_ref[s * _JPB + t]
        r = j - 8 * (j // 8)
        m = (sub == r).astype(jnp.float32)
        o_ref[:, t, :] = jnp.sum(x_refs[t][...] * m, axis=1)


def _make_in_spec(t):
    return pl.BlockSpec(
        (_BATCH, 8, _DIM),
        lambda s, idx_ref, t=t: (0, idx_ref[s * _JPB + t] // 8, 0),
    )


@jax.jit
def _fixgen_gather(idx32, pos):
    grid_spec = pltpu.PrefetchScalarGridSpec(
        num_scalar_prefetch=1,
        grid=(_STEPS,),
        in_specs=[_make_in_spec(t) for t in range(_JPB)],
        out_specs=pl.BlockSpec((_BATCH, _JPB, _DIM), lambda s, idx_ref: (0, s, 0)),
    )
    return pl.pallas_call(
        _gather_body,
        grid_spec=grid_spec,
        out_shape=jax.ShapeDtypeStruct((_BATCH, _NIDX, _DIM), jnp.float32),
    )(idx32, *([pos] * _JPB))


def kernel(pos, idx):
    batch, atm, dim = pos.shape
    idx32 = idx.astype(jnp.int32)
    out3 = _fixgen_gather(idx32, pos)
    return out3.reshape(batch, idx.shape[0] * dim)


# speedup vs baseline: 13.4548x; 1.0070x over previous
import jax
import jax.numpy as jnp
from jax.experimental import pallas as pl
from jax.experimental.pallas import tpu as pltpu

_NIDX = 64


def _gather_body(idx_ref, pos_ref, out_ref, sem):
    def copy(j):
        return pltpu.make_async_copy(
            pos_ref.at[:, pl.ds(idx_ref[j], 1), :],
            out_ref.at[:, pl.ds(j, 1), :],
            sem,
        )

    for j in range(_NIDX):
        copy(j).start()
    for j in range(_NIDX):
        copy(j).wait()


@jax.jit
def _fixgen_gather(idx32, pos):
    batch, _, dim = pos.shape
    return pl.pallas_call(
        _gather_body,
        in_specs=[
            pl.BlockSpec(memory_space=pltpu.MemorySpace.SMEM),
            pl.BlockSpec(memory_space=pltpu.MemorySpace.HBM),
        ],
        out_specs=pl.BlockSpec(memory_space=pltpu.MemorySpace.VMEM),
        out_shape=jax.ShapeDtypeStruct((batch, _NIDX, dim), jnp.float32),
        scratch_shapes=[pltpu.SemaphoreType.DMA],
    )(idx32, pos)


def kernel(pos, idx):
    batch, _, dim = pos.shape
    idx32 = idx.astype(jnp.int32)
    out3 = _fixgen_gather(idx32, pos)
    return out3.reshape(batch, idx.shape[0] * dim)


# speedup vs baseline: 46.9750x; 3.4913x over previous
import jax
import jax.numpy as jnp
from jax.experimental import pallas as pl
from jax.experimental.pallas import tpu as pltpu

_BATCH = 64
_NIDX = 64
_DIM = 3
_FLAT = 300000
_CHUNK = 18816
_STEPS = -(-_FLAT // _CHUNK)


_WIN = 256


def _gather_body(idx_ref, blk_ref, o_ref):
    s = pl.program_id(0)
    base = s * _CHUNK
    for j in range(_NIDX):
        c = idx_ref[j] * _DIM - base
        ok = jnp.logical_and(c >= 0, c < _CHUNK)

        @pl.when(ok)
        def _(c=c, j=j):
            w = jnp.minimum((c // 128) * 128, _CHUNK - _WIN)
            x = blk_ref[:, pl.ds(w, _WIN)]
            r = pltpu.roll(x, _WIN - (c - w), 1)
            o_ref[:, _DIM * j:_DIM * (j + 1)] = r[:, :_DIM]


@jax.jit
def _fixgen_gather(idx32, pos2d):
    grid_spec = pltpu.PrefetchScalarGridSpec(
        num_scalar_prefetch=1,
        grid=(_STEPS,),
        in_specs=[pl.BlockSpec((_BATCH, _CHUNK), lambda s, idx_ref: (0, s))],
        out_specs=pl.BlockSpec((_BATCH, _NIDX * _DIM), lambda s, idx_ref: (0, 0)),
    )
    return pl.pallas_call(
        _gather_body,
        grid_spec=grid_spec,
        out_shape=jax.ShapeDtypeStruct((_BATCH, _NIDX * _DIM), jnp.float32),
    )(idx32, pos2d)


def kernel(pos, idx):
    batch, atm, dim = pos.shape
    idx32 = idx.astype(jnp.int32)
    return _fixgen_gather(idx32, pos.reshape(batch, atm * dim))


# speedup vs baseline: 47.1239x; 1.0032x over previous
import jax
import jax.numpy as jnp
from jax.experimental import pallas as pl
from jax.experimental.pallas import tpu as pltpu

_BATCH = 64
_NIDX = 64
_DIM = 3
_FLAT = 300000
_CHUNK = 100224
_STEPS = -(-_FLAT // _CHUNK)


_WIN = 256


def _gather_body(idx_ref, blk_ref, o_ref):
    s = pl.program_id(0)
    base = s * _CHUNK
    for j in range(_NIDX):
        c = idx_ref[j] * _DIM - base
        ok = jnp.logical_and(c >= 0, c < _CHUNK)

        @pl.when(ok)
        def _(c=c, j=j):
            w = jnp.minimum((c // 128) * 128, _CHUNK - _WIN)
            x = blk_ref[:, pl.ds(w, _WIN)]
            r = pltpu.roll(x, _WIN - (c - w), 1)
            o_ref[:, _DIM * j:_DIM * (j + 1)] = r[:, :_DIM]


@jax.jit
def _fixgen_gather(idx32, pos2d):
    grid_spec = pltpu.PrefetchScalarGridSpec(
        num_scalar_prefetch=1,
        grid=(_STEPS,),
        in_specs=[pl.BlockSpec((_BATCH, _CHUNK), lambda s, idx_ref: (0, s))],
        out_specs=pl.BlockSpec((_BATCH, _NIDX * _DIM), lambda s, idx_ref: (0, 0)),
    )
    return pl.pallas_call(
        _gather_body,
        grid_spec=grid_spec,
        out_shape=jax.ShapeDtypeStruct((_BATCH, _NIDX * _DIM), jnp.float32),
    )(idx32, pos2d)


def kernel(pos, idx):
    batch, atm, dim = pos.shape
    idx32 = idx.astype(jnp.int32)
    return _fixgen_gather(idx32, pos.reshape(batch, atm * dim))
